# Initial kernel scaffold; baseline (speedup 1.0000x reference)
import jax
import jax.numpy as jnp
from jax.experimental import pallas as pl


def kernel(x, edge_index, edge_weight, h0, W_z, b_z, W_r, b_r, W_h, b_h, Lz_W, Lz_b, Lr_W, Lr_b, Lh_W, Lh_b, W_out, b_out):
    raise NotImplementedError("write your pallas kernel here")


# speedup vs baseline: 20.6408x; 20.6408x over previous
import functools

import jax
import jax.numpy as jnp
from jax import lax
from jax.experimental import pallas as pl
from jax.experimental.pallas import tpu as pltpu
from jax.experimental.pallas import tpu_sc as plsc

B = 8
N = 10000
E = 160000
F_IN = 128
D_H = 64
D_D = 7

NC = 2
NS = 16
L = 16
N_PAD = 10240
STRIPE = N_PAD // NS
TE = E // NS
BPC = B // NC
K = 16
NF = F_IN // L


CHUNK = 400
NCHUNK = TE // CHUNK
CB = CHUNK // K
DROW = N_PAD // F_IN
ZROW = DROW // NS


def _sc_agg_kernel(x_hbm, row_hbm, col_hbm, ew_hbm, agg_hbm,
                   normv, degv, disv, dis2sv, idx80, zdeg,
                   rowc, colc, ewc, idxv, cbufv, gbuf,
                   shared_deg, shared_agg):
    c = lax.axis_index("c")
    s = lax.axis_index("s")
    ebase = s * TE

    zero16f = jnp.zeros((L,), jnp.float32)
    zero16i = jnp.zeros((L,), jnp.int32)
    iota16 = lax.iota(jnp.int32, L)

    def zdeg_body(i, _):
        def zf(f, _):
            degv[i, pl.ds(f * L, L)] = zero16f
            return 0
        lax.fori_loop(0, F_IN // L, zf, 0)
        return 0
    lax.fori_loop(0, DROW, zdeg_body, 0)

    def deg_chunk(k, _):
        off = pl.multiple_of(ebase + k * CHUNK, 8)
        pltpu.sync_copy(col_hbm.at[pl.ds(off, CHUNK)], colc)
        pltpu.sync_copy(ew_hbm.at[pl.ds(off, CHUNK)], ewc)

        def deg_body(i, _):
            boff = pl.multiple_of(i * L, 8)
            c16 = colc[pl.ds(boff, L)]
            w16 = ewc[pl.ds(boff, L)]
            plsc.addupdate_scatter(degv, [c16 >> 7, c16 & 127], w16)
            return 0
        lax.fori_loop(0, CHUNK // L, deg_body, 0)
        return 0
    lax.fori_loop(0, NCHUNK, deg_chunk, 0)

    def zd_body(i, _):
        def zf(f, _):
            zdeg[i, pl.ds(f * L, L)] = zero16f
            return 0
        lax.fori_loop(0, F_IN // L, zf, 0)
        return 0
    lax.fori_loop(0, ZROW, zd_body, 0)

    def i80_body(i, _):
        off = pl.multiple_of(i * L, 8)
        idx80[pl.ds(off, L)] = iota16 + i * L
        return 0
    lax.fori_loop(0, DROW // L, i80_body, 0)

    pltpu.sync_copy(zdeg, shared_deg.at[pl.ds(s * ZROW, ZROW)])
    plsc.subcore_barrier()
    pltpu.sync_copy(degv, shared_deg.at[idx80], add=True)
    plsc.subcore_barrier()
    pltpu.sync_copy(shared_deg, degv)

    magic = zero16i + jnp.int32(0x5F3759DF)

    def dis_body(i, _):
        dg = degv[i >> 3, pl.ds(pl.multiple_of((i & 7) * L, 16), L)] + 2.0
        y = plsc.bitcast(magic - (plsc.bitcast(dg, jnp.int32) >> 1),
                         jnp.float32)
        for _it in range(3):
            y = y * (1.5 - 0.5 * dg * y * y)
        off = pl.multiple_of(i * L, 8)
        disv[pl.ds(off, L)] = y
        return 0
    lax.fori_loop(0, N_PAD // L, dis_body, 0)

    def dis2_body(i, _):
        off = pl.multiple_of(i * L, 8)
        y = disv[pl.ds(s * STRIPE + off, L)]
        dis2sv[pl.ds(off, L)] = 2.0 * y * y
        return 0
    lax.fori_loop(0, STRIPE // L, dis2_body, 0)

    def norm_chunk(k, _):
        off = pl.multiple_of(ebase + k * CHUNK, 8)
        pltpu.sync_copy(row_hbm.at[pl.ds(off, CHUNK)], rowc)
        pltpu.sync_copy(col_hbm.at[pl.ds(off, CHUNK)], colc)
        pltpu.sync_copy(ew_hbm.at[pl.ds(off, CHUNK)], ewc)

        def norm_body(i, _):
            boff = pl.multiple_of(i * L, 8)
            r16 = rowc[pl.ds(boff, L)]
            c16 = colc[pl.ds(boff, L)]
            w16 = ewc[pl.ds(boff, L)]
            dr = plsc.load_gather(disv, [r16])
            dc = plsc.load_gather(disv, [c16])
            noff = pl.multiple_of(k * CHUNK + i * L, 8)
            normv[pl.ds(noff, L)] = dr * w16 * dc
            return 0
        lax.fori_loop(0, CHUNK // L, norm_body, 0)
        return 0
    lax.fori_loop(0, NCHUNK, norm_chunk, 0)

    nvalid = jnp.minimum(STRIPE, N - s * STRIPE)

    def batch_body(b, _):
        gb = c * BPC + b

        def init_body(j, _):
            d0 = s * STRIPE + j * K
            src = pl.multiple_of(gb * N + d0, 8)
            pltpu.sync_copy(x_hbm.at[pl.ds(src, K)], gbuf)

            def scale_i(e, _):
                dsp = plsc.load_gather(dis2sv, [zero16i + (j * K + e)])
                for f in range(NF):
                    gbuf[e, pl.ds(f * L, L)] = gbuf[e, pl.ds(f * L, L)] * dsp
                return 0
            lax.fori_loop(0, K, scale_i, 0)
            dst = pl.multiple_of(d0, 8)
            pltpu.sync_copy(gbuf, shared_agg.at[pl.ds(dst, K)])
            return 0
        lax.fori_loop(0, nvalid // K, init_body, 0)
        plsc.subcore_barrier()

        def edge_chunk(k, _):
            eoff = pl.multiple_of(ebase + k * CHUNK, 8)
            pltpu.sync_copy(row_hbm.at[pl.ds(eoff, CHUNK)], rowc)
            pltpu.sync_copy(col_hbm.at[pl.ds(eoff, CHUNK)], colc)

            def edge_body(i, _):
                boff = pl.multiple_of(i * K, 8)
                idxv[...] = rowc[pl.ds(boff, K)] + gb * N
                cbufv[...] = colc[pl.ds(boff, K)]
                pltpu.sync_copy(x_hbm.at[idxv], gbuf)

                def scale_e(e, _):
                    nsp = plsc.load_gather(
                        normv, [zero16i + (k * CHUNK + boff + e)])
                    for f in range(NF):
                        gbuf[e, pl.ds(f * L, L)] = \
                            gbuf[e, pl.ds(f * L, L)] * nsp
                    return 0
                lax.fori_loop(0, K, scale_e, 0)
                pltpu.sync_copy(gbuf, shared_agg.at[cbufv], add=True)
                return 0
            lax.fori_loop(0, CB, edge_body, 0)
            return 0
        lax.fori_loop(0, NCHUNK, edge_chunk, 0)
        plsc.subcore_barrier()

        def wb_body(j, _):
            so = pl.multiple_of(s * STRIPE + j * K, 8)
            wo = pl.multiple_of(gb * N_PAD + s * STRIPE + j * K, 8)
            pltpu.sync_copy(shared_agg.at[pl.ds(so, K)], gbuf)
            pltpu.sync_copy(gbuf, agg_hbm.at[pl.ds(wo, K)])
            return 0
        lax.fori_loop(0, STRIPE // K, wb_body, 0)
        plsc.subcore_barrier()
        return 0
    lax.fori_loop(0, BPC, batch_body, 0)


def _sc_aggregate(x_flat, row, col, ew):
    mesh = plsc.VectorSubcoreMesh(core_axis_name="c", subcore_axis_name="s")
    f = pl.kernel(
        _sc_agg_kernel,
        out_type=jax.ShapeDtypeStruct((B * N_PAD, F_IN), jnp.float32),
        mesh=mesh,
        compiler_params=pltpu.CompilerParams(
            needs_layout_passes=False,
            internal_scratch_in_bytes=32 * 1024,
        ),
        scratch_types=[
            pltpu.VMEM((TE,), jnp.float32),
            pltpu.VMEM((DROW, F_IN), jnp.float32),
            pltpu.VMEM((N_PAD,), jnp.float32),
            pltpu.VMEM((STRIPE,), jnp.float32),
            pltpu.VMEM((DROW,), jnp.int32),
            pltpu.VMEM((ZROW, F_IN), jnp.float32),
            pltpu.VMEM((CHUNK,), jnp.int32),
            pltpu.VMEM((CHUNK,), jnp.int32),
            pltpu.VMEM((CHUNK,), jnp.float32),
            pltpu.VMEM((K,), jnp.int32),
            pltpu.VMEM((K,), jnp.int32),
            pltpu.VMEM((K, F_IN), jnp.float32),
            pltpu.VMEM_SHARED((DROW, F_IN), jnp.float32),
            pltpu.VMEM_SHARED((N_PAD, F_IN), jnp.float32),
        ],
    )
    return f(x_flat, row, col, ew)


def _tc_dense_kernel(agg_ref, h0_ref,
                     Wz_ref, bz_ref, Wr_ref, br_ref, Wh_ref, bh_ref,
                     LzW_ref, Lzb_ref, LrW_ref, Lrb_ref, LhW_ref, Lhb_ref,
                     Wo_ref, bo_ref, H_ref, y_ref):
    f32 = jnp.float32
    a = agg_ref[: N, :]
    h0 = h0_ref[0]

    LzT, LzB = LzW_ref[:D_H], LzW_ref[D_H:]
    LrT, LrB = LrW_ref[:D_H], LrW_ref[D_H:]
    LhT, LhB = LhW_ref[:D_H], LhW_ref[D_H:]

    Mz = jnp.dot(Wz_ref[...], LzT, preferred_element_type=f32)
    Mr = jnp.dot(Wr_ref[...], LrT, preferred_element_type=f32)
    Mh = jnp.dot(Wh_ref[...], LhT, preferred_element_type=f32)

    cz = jnp.dot(bz_ref[...], LzT, preferred_element_type=f32) + Lzb_ref[...]
    cr = jnp.dot(br_ref[...], LrT, preferred_element_type=f32) + Lrb_ref[...]
    ch = jnp.dot(bh_ref[...], LhT, preferred_element_type=f32) + Lhb_ref[...]

    Z = jax.nn.sigmoid(jnp.dot(a, Mz, preferred_element_type=f32)
                       + jnp.dot(h0, LzB, preferred_element_type=f32) + cz)
    R = jax.nn.sigmoid(jnp.dot(a, Mr, preferred_element_type=f32)
                       + jnp.dot(h0, LrB, preferred_element_type=f32) + cr)
    Ht = jnp.tanh(jnp.dot(a, Mh, preferred_element_type=f32)
                  + jnp.dot(h0 * R, LhB, preferred_element_type=f32) + ch)
    H = Z * h0 + (1.0 - Z) * Ht
    y = jnp.dot(jax.nn.relu(H), Wo_ref[...], preferred_element_type=f32) \
        + bo_ref[...]
    H_ref[0] = H
    y_ref[0] = y


def _tc_dense(agg_flat, h0, W_z, b_z, W_r, b_r, W_h, b_h,
              Lz_W, Lz_b, Lr_W, Lr_b, Lh_W, Lh_b, W_out, b_out):
    full = lambda shape: pl.BlockSpec(shape, lambda i: (0,) * len(shape))
    grid_spec = pl.GridSpec(
        grid=(B,),
        in_specs=[
            pl.BlockSpec((N_PAD, F_IN), lambda i: (i, 0)),
            pl.BlockSpec((1, N, D_H), lambda i: (i, 0, 0)),
            full((F_IN, D_H)), full((1, D_H)),
            full((F_IN, D_H)), full((1, D_H)),
            full((F_IN, D_H)), full((1, D_H)),
            full((2 * D_H, D_H)), full((1, D_H)),
            full((2 * D_H, D_H)), full((1, D_H)),
            full((2 * D_H, D_H)), full((1, D_H)),
            full((D_H, D_D)), full((1, D_D)),
        ],
        out_specs=[
            pl.BlockSpec((1, N, D_H), lambda i: (i, 0, 0)),
            pl.BlockSpec((1, N, D_D), lambda i: (i, 0, 0)),
        ],
    )
    return pl.pallas_call(
        _tc_dense_kernel,
        grid_spec=grid_spec,
        out_shape=[
            jax.ShapeDtypeStruct((B, N, D_H), jnp.float32),
            jax.ShapeDtypeStruct((B, N, D_D), jnp.float32),
        ],
    )(agg_flat, h0, W_z, b_z[None, :], W_r, b_r[None, :], W_h, b_h[None, :],
      Lz_W, Lz_b[None, :], Lr_W, Lr_b[None, :], Lh_W, Lh_b[None, :],
      W_out, b_out[None, :])


def kernel(x, edge_index, edge_weight, h0,
           W_z, b_z, W_r, b_r, W_h, b_h,
           Lz_W, Lz_b, Lr_W, Lr_b, Lh_W, Lh_b,
           W_out, b_out):
    x_flat = x.reshape(B * N, F_IN)
    row = edge_index[0]
    col = edge_index[1]
    agg_flat = _sc_aggregate(x_flat, row, col, edge_weight)
    H, y = _tc_dense(agg_flat, h0, W_z, b_z, W_r, b_r, W_h, b_h,
                     Lz_W, Lz_b, Lr_W, Lr_b, Lh_W, Lh_b, W_out, b_out)
    return (H, y)
